# Initial kernel scaffold; baseline (speedup 1.0000x reference)
import jax
import jax.numpy as jnp
from jax.experimental import pallas as pl


def kernel(x, pos, edge_attr, edge_index, batch, Wn, bn, We, be, W1, b1, W2, b2):
    raise NotImplementedError("write your pallas kernel here")


# speedup vs baseline: 1.6499x; 1.6499x over previous
import functools

import jax
import jax.numpy as jnp
from jax import lax
from jax.experimental import pallas as pl
from jax.experimental.pallas import tpu as pltpu
from jax.experimental.pallas import tpu_sc as plsc

N = 10000
E = 320000
XD = 128
PD = 3
ED = 16
H = 256
HH = 128
L = 4

NSUB = 16
CHUNK = 128
EPW = 20480
NCH = EPW // CHUNK
NGRP = NCH // 8
E_PAD = EPW * NSUB
DUMMY = N
AGG_ROWS = 10112
NZCH = AGG_ROWS // CHUNK

ROW_BLK = 400
EB_BLK = 512


def _enc_body(f_ref, w_ref, b_ref, o_ref):
    o_ref[0] = (
        jnp.dot(f_ref[...], w_ref[...], preferred_element_type=jnp.float32)
        + b_ref[...]
    )


def _encode(feats, w, b, n_rows, row_blk):
    k = feats.shape[1]
    grid = (2, n_rows // row_blk)
    return pl.pallas_call(
        _enc_body,
        grid=grid,
        in_specs=[
            pl.BlockSpec((row_blk, k), lambda i, j: (j, 0)),
            pl.BlockSpec((k, HH), lambda i, j: (0, i)),
            pl.BlockSpec((HH,), lambda i, j: (i,)),
        ],
        out_specs=pl.BlockSpec((1, row_blk, HH), lambda i, j: (i, j, 0)),
        out_shape=jax.ShapeDtypeStruct((2, n_rows, HH), jnp.float32),
    )(feats, w, b)


def _mlp_body(agg_ref, h_ref, w1_ref, b1_ref, w2_ref, b2_ref, o_ref):
    a = jnp.concatenate([agg_ref[0], agg_ref[1]], axis=1)
    t = jnp.maximum(
        jnp.dot(a, w1_ref[...], preferred_element_type=jnp.float32)
        + b1_ref[...],
        0.0,
    )
    o = jnp.dot(t, w2_ref[...], preferred_element_type=jnp.float32) + b2_ref[...]
    o_ref[0] = o[:, :HH] + h_ref[0]
    o_ref[1] = o[:, HH:] + h_ref[1]


def _mlp(agg, h, w1, b1, w2, b2):
    grid = (N // ROW_BLK,)
    return pl.pallas_call(
        _mlp_body,
        grid=grid,
        in_specs=[
            pl.BlockSpec((2, ROW_BLK, HH), lambda j: (0, j, 0)),
            pl.BlockSpec((2, ROW_BLK, HH), lambda j: (0, j, 0)),
            pl.BlockSpec((H, H), lambda j: (0, 0)),
            pl.BlockSpec((H,), lambda j: (0,)),
            pl.BlockSpec((H, H), lambda j: (0, 0)),
            pl.BlockSpec((H,), lambda j: (0,)),
        ],
        out_specs=pl.BlockSpec((2, ROW_BLK, HH), lambda j: (0, j, 0)),
        out_shape=jax.ShapeDtypeStruct((2, N, HH), jnp.float32),
    )(agg, h, w1, b1, w2, b2)


def _sc_body(h_hbm, ea_hbm, src_hbm, dst_hbm, out_hbm,
             srcv, dstv, hbuf, ebuf, aggs):
    c = lax.axis_index("c")
    s = lax.axis_index("s")

    zero = jnp.zeros((16,), jnp.float32)

    @pl.loop(0, CHUNK)
    def _zrow(i):
        for g in range(HH // 16):
            ebuf[i, pl.ds(g * 16, 16)] = zero

    @pl.loop(0, 5)
    def _zagg(k):
        zc = s + k * NSUB

        @pl.when(zc < NZCH)
        def _do():
            pltpu.sync_copy(ebuf, aggs.at[pl.ds(zc * CHUNK, CHUNK)])

    plsc.subcore_barrier()

    @pl.loop(0, NGRP)
    def _grp(g):
        pltpu.sync_copy(src_hbm.at[s, pl.ds(g * 8, 8)], srcv)
        pltpu.sync_copy(dst_hbm.at[s, pl.ds(g * 8, 8)], dstv)
        for jj in range(8):
            e0 = s * EPW + g * (8 * CHUNK) + jj * CHUNK
            pltpu.sync_copy(ea_hbm.at[c, pl.ds(e0, CHUNK)], ebuf)
            pltpu.sync_copy(h_hbm.at[c].at[srcv.at[jj]], hbuf)

            @pl.loop(0, CHUNK)
            def _row(i):
                for q in range(HH // 16):
                    sl = pl.ds(q * 16, 16)
                    hbuf[i, sl] = jnp.maximum(hbuf[i, sl] + ebuf[i, sl], 0.0)

            pltpu.sync_copy(hbuf, aggs.at[dstv.at[jj]], add=True)

    plsc.subcore_barrier()

    pltpu.sync_copy(aggs.at[pl.ds(s * 624, 624)],
                    out_hbm.at[c, pl.ds(s * 624, 624)])

    @pl.when(s == 0)
    def _tail():
        pltpu.sync_copy(aggs.at[pl.ds(9984, 16)],
                        out_hbm.at[c, pl.ds(9984, 16)])


def _sc_edge_phase(h_stack, ea_stack, src_r, dst_r):
    mesh = plsc.VectorSubcoreMesh(core_axis_name="c", subcore_axis_name="s")
    kern = pl.kernel(
        _sc_body,
        out_type=jax.ShapeDtypeStruct((2, N, HH), jnp.float32),
        mesh=mesh,
        scratch_types=[
            pltpu.VMEM((8, CHUNK), jnp.int32),
            pltpu.VMEM((8, CHUNK), jnp.int32),
            pltpu.VMEM((CHUNK, HH), jnp.float32),
            pltpu.VMEM((CHUNK, HH), jnp.float32),
            pltpu.VMEM_SHARED((AGG_ROWS, HH), jnp.float32),
        ],
    )
    return kern(h_stack, ea_stack, src_r, dst_r)


def kernel(x, pos, edge_attr, edge_index, batch, Wn, bn, We, be, W1, b1, W2, b2):
    del batch
    feats = jnp.concatenate(
        [x, pos, jnp.zeros((N, H - XD - PD), jnp.float32)], axis=1)
    wn_p = jnp.concatenate(
        [Wn, jnp.zeros((H - XD - PD, H), jnp.float32)], axis=0)
    ea_in = jnp.concatenate(
        [edge_attr, jnp.zeros((E_PAD - E, ED), jnp.float32)], axis=0)
    src = jnp.concatenate(
        [edge_index[0], jnp.zeros((E_PAD - E,), jnp.int32)]).reshape(
            NSUB, NCH, CHUNK)
    dst = jnp.concatenate(
        [edge_index[1], jnp.full((E_PAD - E,), DUMMY, jnp.int32)]).reshape(
            NSUB, NCH, CHUNK)

    h = _encode(feats, wn_p, bn, N, ROW_BLK)
    ea = _encode(ea_in, We, be, E_PAD, EB_BLK)

    for i in range(L):
        agg = _sc_edge_phase(h, ea, src, dst)
        h = _mlp(agg, h, W1[i], b1[i], W2[i], b2[i])

    return jnp.concatenate([h[0], h[1]], axis=1)


# speedup vs baseline: 1.8868x; 1.1436x over previous
import functools

import jax
import jax.numpy as jnp
from jax import lax
from jax.experimental import pallas as pl
from jax.experimental.pallas import tpu as pltpu
from jax.experimental.pallas import tpu_sc as plsc

N = 10000
E = 320000
XD = 128
PD = 3
ED = 16
H = 256
HH = 128
L = 4

NSUB = 16
CHUNK = 64
GCH = 6
NGRP = 54
NCH = GCH * NGRP
EPW = NCH * CHUNK
E_PAD = EPW * NSUB
DUMMY = N
AGG_ROWS = 10240

ROW_BLK = 400
EB_BLK = 512


def _enc_body(f_ref, w_ref, b_ref, o_ref):
    o_ref[0] = (
        jnp.dot(f_ref[...], w_ref[...], preferred_element_type=jnp.float32)
        + b_ref[...]
    )


def _encode(feats, w, b, n_rows, row_blk):
    k = feats.shape[1]
    grid = (2, n_rows // row_blk)
    return pl.pallas_call(
        _enc_body,
        grid=grid,
        in_specs=[
            pl.BlockSpec((row_blk, k), lambda i, j: (j, 0)),
            pl.BlockSpec((k, HH), lambda i, j: (0, i)),
            pl.BlockSpec((HH,), lambda i, j: (i,)),
        ],
        out_specs=pl.BlockSpec((1, row_blk, HH), lambda i, j: (i, j, 0)),
        out_shape=jax.ShapeDtypeStruct((2, n_rows, HH), jnp.float32),
    )(feats, w, b)


def _mlp_body(agg_ref, h_ref, w1_ref, b1_ref, w2_ref, b2_ref, o_ref):
    a = jnp.concatenate([agg_ref[0], agg_ref[1]], axis=1)
    t = jnp.maximum(
        jnp.dot(a, w1_ref[...], preferred_element_type=jnp.float32)
        + b1_ref[...],
        0.0,
    )
    o = jnp.dot(t, w2_ref[...], preferred_element_type=jnp.float32) + b2_ref[...]
    o_ref[0] = o[:, :HH] + h_ref[0]
    o_ref[1] = o[:, HH:] + h_ref[1]


def _mlp(agg, h, w1, b1, w2, b2):
    grid = (N // ROW_BLK,)
    return pl.pallas_call(
        _mlp_body,
        grid=grid,
        in_specs=[
            pl.BlockSpec((2, ROW_BLK, HH), lambda j: (0, j, 0)),
            pl.BlockSpec((2, ROW_BLK, HH), lambda j: (0, j, 0)),
            pl.BlockSpec((H, H), lambda j: (0, 0)),
            pl.BlockSpec((H,), lambda j: (0,)),
            pl.BlockSpec((H, H), lambda j: (0, 0)),
            pl.BlockSpec((H,), lambda j: (0,)),
        ],
        out_specs=pl.BlockSpec((2, ROW_BLK, HH), lambda j: (0, j, 0)),
        out_shape=jax.ShapeDtypeStruct((2, N, HH), jnp.float32),
    )(agg, h, w1, b1, w2, b2)


def _sc_body(h_hbm, ea_hbm, src_hbm, dst_hbm, out_hbm,
             srcv, dstv, hbuf, ebuf, aggs, gsem, esem, ssem, isem):
    c = lax.axis_index("c")
    s = lax.axis_index("s")

    zero = jnp.zeros((16,), jnp.float32)

    @pl.loop(0, CHUNK)
    def _zrow(i):
        for q in range(HH // 16):
            ebuf[0, i, pl.ds(q * 16, 16)] = zero

    @pl.loop(0, 10)
    def _zagg(k):
        pltpu.sync_copy(ebuf.at[0],
                        aggs.at[pl.ds((s + k * NSUB) * CHUNK, CHUNK)])

    plsc.subcore_barrier()

    def wait_gather(p):
        pltpu.make_async_copy(
            h_hbm.at[c, pl.ds(0, CHUNK)], hbuf.at[p], gsem.at[p]).wait()

    def wait_ea(p):
        pltpu.make_async_copy(
            ea_hbm.at[c, pl.ds(0, CHUNK)], ebuf.at[p], esem.at[p]).wait()

    def wait_scatter(p):
        pltpu.make_async_copy(
            hbuf.at[p], aggs.at[pl.ds(0, CHUNK)], ssem.at[p]).wait()

    pltpu.sync_copy(src_hbm.at[s, 0], srcv.at[0])
    pltpu.sync_copy(dst_hbm.at[s, 0], dstv.at[0])
    for b in range(2):
        pltpu.async_copy(
            ea_hbm.at[c, pl.ds(s * EPW + b * CHUNK, CHUNK)],
            ebuf.at[b], esem.at[b])
        pltpu.async_copy(
            h_hbm.at[c].at[srcv.at[0, b]], hbuf.at[b], gsem.at[b])

    @pl.loop(0, NGRP // 2)
    def _pair(t):
        for gg in range(2):
            g = 2 * t + gg
            for jj in range(GCH):
                k = g * GCH + jj
                p3 = jj % 3
                p2 = jj % 2
                q3 = (jj + 2) % 3

                wait_gather(p3)
                wait_ea(p2)

                @plsc.parallel_loop(0, CHUNK, unroll=2)
                def _row(i):
                    for q in range(HH // 16):
                        sl = pl.ds(q * 16, 16)
                        hbuf[p3, i, sl] = jnp.maximum(
                            hbuf[p3, i, sl] + ebuf[p2, i, sl], 0.0)

                pltpu.async_copy(hbuf.at[p3], aggs.at[dstv.at[gg, jj]],
                                 ssem.at[p3], add=True)

                @pl.when(k + 2 < NCH)
                def _ea_next():
                    pltpu.async_copy(
                        ea_hbm.at[c, pl.ds(s * EPW + (k + 2) * CHUNK, CHUNK)],
                        ebuf.at[p2], esem.at[p2])

                @pl.when(k >= 1)
                def _ws():
                    wait_scatter(q3)

                if jj == 0:
                    @pl.when(g + 1 < NGRP)
                    def _ipf():
                        pltpu.async_copy(src_hbm.at[s, g + 1],
                                         srcv.at[1 - gg], isem.at[0])
                        pltpu.async_copy(dst_hbm.at[s, g + 1],
                                         dstv.at[1 - gg], isem.at[1])

                if jj == 4:
                    @pl.when(g + 1 < NGRP)
                    def _iw():
                        pltpu.make_async_copy(
                            src_hbm.at[s, 0], srcv.at[1 - gg],
                            isem.at[0]).wait()
                        pltpu.make_async_copy(
                            dst_hbm.at[s, 0], dstv.at[1 - gg],
                            isem.at[1]).wait()

                @pl.when(k + 2 < NCH)
                def _g_next():
                    if jj < 4:
                        sidx = srcv.at[gg, jj + 2]
                    else:
                        sidx = srcv.at[1 - gg, jj - 4]
                    pltpu.async_copy(h_hbm.at[c].at[sidx], hbuf.at[q3],
                                     gsem.at[q3])

    wait_scatter((NCH - 1) % 3)

    plsc.subcore_barrier()

    pltpu.sync_copy(aggs.at[pl.ds(s * 624, 624)],
                    out_hbm.at[c, pl.ds(s * 624, 624)])

    @pl.when(s == 0)
    def _tail():
        pltpu.sync_copy(aggs.at[pl.ds(9984, 16)],
                        out_hbm.at[c, pl.ds(9984, 16)])


def _sc_edge_phase(h_stack, ea_stack, src_r, dst_r):
    mesh = plsc.VectorSubcoreMesh(core_axis_name="c", subcore_axis_name="s")
    kern = pl.kernel(
        _sc_body,
        out_type=jax.ShapeDtypeStruct((2, N, HH), jnp.float32),
        mesh=mesh,
        scratch_types=[
            pltpu.VMEM((2, GCH, CHUNK), jnp.int32),
            pltpu.VMEM((2, GCH, CHUNK), jnp.int32),
            pltpu.VMEM((3, CHUNK, HH), jnp.float32),
            pltpu.VMEM((2, CHUNK, HH), jnp.float32),
            pltpu.VMEM_SHARED((AGG_ROWS, HH), jnp.float32),
            pltpu.SemaphoreType.DMA((3,)),
            pltpu.SemaphoreType.DMA((2,)),
            pltpu.SemaphoreType.DMA((3,)),
            pltpu.SemaphoreType.DMA((2,)),
        ],
    )
    return kern(h_stack, ea_stack, src_r, dst_r)


def kernel(x, pos, edge_attr, edge_index, batch, Wn, bn, We, be, W1, b1, W2, b2):
    del batch
    feats = jnp.concatenate(
        [x, pos, jnp.zeros((N, H - XD - PD), jnp.float32)], axis=1)
    wn_p = jnp.concatenate(
        [Wn, jnp.zeros((H - XD - PD, H), jnp.float32)], axis=0)
    ea_in = jnp.concatenate(
        [edge_attr, jnp.zeros((E_PAD - E, ED), jnp.float32)], axis=0)
    src = jnp.concatenate(
        [edge_index[0], jnp.zeros((E_PAD - E,), jnp.int32)]).reshape(
            NSUB, NGRP, GCH, CHUNK)
    dst = jnp.concatenate(
        [edge_index[1], jnp.full((E_PAD - E,), DUMMY, jnp.int32)]).reshape(
            NSUB, NGRP, GCH, CHUNK)

    h = _encode(feats, wn_p, bn, N, ROW_BLK)
    ea = _encode(ea_in, We, be, E_PAD, EB_BLK)

    for i in range(L):
        agg = _sc_edge_phase(h, ea, src, dst)
        h = _mlp(agg, h, W1[i], b1[i], W2[i], b2[i])

    return jnp.concatenate([h[0], h[1]], axis=1)
